# baseline (speedup 1.0000x reference)
import functools

import jax
import jax.numpy as jnp
from jax import lax
from jax.experimental import pallas as pl
from jax.experimental.pallas import tpu as pltpu
from jax.experimental.pallas import tpu_sc as plsc

N = 10000
E = 160000
U = 64
IN_DIM = 2
K = 2
B = 8
IN_SIZE = IN_DIM + U
NUM_MAT = 2 * K + 1
FAN_IN = IN_SIZE * NUM_MAT
FAN_PAD = 384
MBLK = 800

NC = 2
NS = 16
NW = NC * NS
L = 16
NRT = 320
N_PAD = NW * NRT
FH = 272
HALF = 264
CH = 64
META = 2048
E_PAD = E + 2 * META
ROW_SENTINEL = 1 << 20


def _splits_at(splits_v, w):
    i = lax.iota(jnp.int32, L)
    total = jnp.zeros((L,), jnp.int32)
    for cidx in range(48 // L):
        ch = splits_v[pl.ds(cidx * L, L)]
        lane = w - cidx * L
        total = total + jnp.where(i == lane, ch, 0)
    return jnp.sum(total)


def _spmm_body(xa, xb, meta, splits, ya, yb,
               splits_v, mbuf, gbuf0, gbuf1, acc,
               gsem0, gsem1):
    c = lax.axis_index("c")
    s = lax.axis_index("s")
    wid = s * NC + c
    pltpu.sync_copy(splits, splits_v)
    start = _splits_at(splits_v, wid)
    end = _splits_at(splits_v, wid + 1)
    base = wid * NRT
    a_start = (start // 8) * 8
    nbig = (end - a_start + META - 1) // META
    NPAIR = META // (2 * CH)
    LASTA = (NPAIR - 1) * 2 * CH

    for tab, out in ((xa, ya), (xb, yb)):
        @plsc.parallel_loop(0, NRT, unroll=2)
        def zero_body(i):
            for j in range(FH // L):
                acc[i, pl.ds(j * L, L)] = jnp.zeros((L,), jnp.float32)

        def proc(gb, moff):
            @plsc.parallel_loop(0, CH // L, unroll=1)
            def grp_body(g16):
                m = moff + g16 * L
                rows16 = mbuf[0, pl.ds(m, L)]
                vals16 = plsc.bitcast(mbuf[2, pl.ds(m, L)], jnp.float32)
                ok16 = (rows16 >= base) & (rows16 < base + NRT)
                v16 = jnp.where(ok16, vals16, 0.0)
                lr16 = lax.max(0, lax.min(rows16 - base, NRT - 1))
                for e in range(L):
                    v = v16[e]
                    lr = lr16[e]
                    eb = g16 * L + e
                    xs = [gb[eb, pl.ds(j * L, L)] for j in range(FH // L)]
                    ps = [v * x for x in xs]
                    for j in range(FH // L):
                        plsc.addupdate(acc.at[lr, pl.ds(j * L, L)], ps[j])

        def drain0():
            pltpu.make_async_copy(tab.at[pl.ds(0, CH)], gbuf0, gsem0).wait()

        def drain1():
            pltpu.make_async_copy(tab.at[pl.ds(0, CH)], gbuf1, gsem1).wait()

        def big_body(g, _):
            off = a_start + g * META
            pltpu.sync_copy(meta.at[:, pl.ds(off, META)], mbuf)
            pltpu.async_copy(tab.at[mbuf.at[1, pl.ds(0, CH)]], gbuf0, gsem0)
            pltpu.async_copy(tab.at[mbuf.at[1, pl.ds(CH, CH)]], gbuf1, gsem1)

            def pair_body(p, _):
                moffA = p * 2 * CH
                nextA = lax.min(moffA + 2 * CH, LASTA)
                drain0()
                proc(gbuf0, moffA)
                pltpu.async_copy(
                    tab.at[mbuf.at[1, pl.ds(nextA, CH)]], gbuf0, gsem0)
                drain1()
                proc(gbuf1, moffA + CH)
                pltpu.async_copy(
                    tab.at[mbuf.at[1, pl.ds(nextA + CH, CH)]], gbuf1, gsem1)
                return 0
            lax.fori_loop(0, NPAIR, pair_body, 0)
            drain0()
            drain1()
            return 0
        lax.fori_loop(0, nbig, big_body, 0)
        pltpu.sync_copy(acc, out.at[pl.ds(base, NRT)])


_sc_spmm = pl.kernel(
    _spmm_body,
    out_type=[jax.ShapeDtypeStruct((N_PAD, FH), jnp.float32)] * 2,
    mesh=plsc.VectorSubcoreMesh(core_axis_name="c", subcore_axis_name="s"),
    compiler_params=pltpu.CompilerParams(
        needs_layout_passes=False, use_tc_tiling_on_sc=False),
    scratch_types=[
        pltpu.VMEM((48,), jnp.int32),
        pltpu.VMEM((3, META), jnp.int32),
        pltpu.VMEM((CH, FH), jnp.float32),
        pltpu.VMEM((CH, FH), jnp.float32),
        pltpu.VMEM((NRT, FH), jnp.float32),
        pltpu.SemaphoreType.DMA,
        pltpu.SemaphoreType.DMA,
    ],
)


def _prep_support(rows, cols, vals):
    order = jnp.argsort(rows)
    rs = rows[order]
    cs = cols[order]
    vs = vals[order]
    bounds = NRT * jnp.arange(NW + 1, dtype=jnp.int32)
    splits = jnp.searchsorted(rs, bounds).astype(jnp.int32)
    splits = jnp.pad(splits, (0, 48 - (NW + 1)))
    rs = jnp.pad(rs, (0, E_PAD - E), constant_values=ROW_SENTINEL)
    cs = jnp.pad(cs, (0, E_PAD - E))
    vs = jnp.pad(vs, (0, E_PAD - E))
    meta = jnp.stack([rs, cs, vs.view(jnp.int32)], axis=0)
    return meta, splits


def _split_x0(x0):
    xa = jnp.pad(x0[:, :HALF], ((0, N_PAD - N), (0, FH - HALF)))
    xb = jnp.pad(x0[:, HALF:], ((0, N_PAD - N), (0, FH - HALF)))
    return xa, xb


def _cheb_stack_sc(x0a, x0b, sup1, sup2):
    xs = [(x0a, x0b)]
    x0 = (x0a, x0b)
    for sup in (sup1, sup2):
        x1 = _sc_spmm(x0[0], x0[1], *sup)
        xs.append(x1)
        z = _sc_spmm(x1[0], x1[1], *sup)
        x2 = (2.0 * z[0] - x0[0], 2.0 * z[1] - x0[1])
        xs.append(x2)
        x1, x0 = x2, x1
    return xs


def _to_xarr_sc(xs):
    mats = [jnp.concatenate([a[:N, :HALF], b[:N, :HALF]], axis=1)
            for (a, b) in xs]
    xarr = jnp.stack(mats, axis=0).reshape(NUM_MAT, N, IN_SIZE, B)
    return jnp.transpose(xarr, (3, 1, 2, 0)).reshape(B * N, FAN_IN)


def _ru_body(x_ref, w_ref, b_ref, r_ref, u_ref):
    acc = jnp.dot(x_ref[...], w_ref[...], preferred_element_type=jnp.float32)
    val = jax.nn.sigmoid(acc + b_ref[...])
    r_ref[...] = val[:, :U]
    u_ref[...] = val[:, U:]


def _gru_body(x_ref, w_ref, b_ref, u_ref, hx_ref, out_ref):
    acc = jnp.dot(x_ref[...], w_ref[...], preferred_element_type=jnp.float32)
    c = jnp.tanh(acc + b_ref[...])
    u = u_ref[...]
    out_ref[...] = u * hx_ref[...] + (1.0 - u) * c


def _ru_call(xarr, W, b):
    grid = (B * N) // MBLK
    return pl.pallas_call(
        _ru_body,
        grid=(grid,),
        in_specs=[
            pl.BlockSpec((MBLK, FAN_IN), lambda i: (i, 0)),
            pl.BlockSpec((FAN_IN, 2 * U), lambda i: (0, 0)),
            pl.BlockSpec((1, 2 * U), lambda i: (0, 0)),
        ],
        out_specs=[
            pl.BlockSpec((MBLK, U), lambda i: (i, 0)),
            pl.BlockSpec((MBLK, U), lambda i: (i, 0)),
        ],
        out_shape=[
            jax.ShapeDtypeStruct((B * N, U), jnp.float32),
            jax.ShapeDtypeStruct((B * N, U), jnp.float32),
        ],
    )(xarr, W, b.reshape(1, -1))


def _gru_call(xarr, W, b, u, hx):
    grid = (B * N) // MBLK
    return pl.pallas_call(
        _gru_body,
        grid=(grid,),
        in_specs=[
            pl.BlockSpec((MBLK, FAN_IN), lambda i: (i, 0)),
            pl.BlockSpec((FAN_IN, U), lambda i: (0, 0)),
            pl.BlockSpec((1, U), lambda i: (0, 0)),
            pl.BlockSpec((MBLK, U), lambda i: (i, 0)),
            pl.BlockSpec((MBLK, U), lambda i: (i, 0)),
        ],
        out_specs=pl.BlockSpec((MBLK, U), lambda i: (i, 0)),
        out_shape=jax.ShapeDtypeStruct((B * N, U), jnp.float32),
    )(xarr, W, b.reshape(1, -1), u, hx)


def kernel(inputs, hx, W_ru, b_ru, W_c, b_c,
           s1_rows, s1_cols, s1_vals, s2_rows, s2_cols, s2_vals):
    sup1 = _prep_support(s1_rows, s1_cols, s1_vals)
    sup2 = _prep_support(s2_rows, s2_cols, s2_vals)
    W_ru_p = W_ru
    W_c_p = W_c

    inp3 = inputs.reshape(B, N, IN_DIM)
    hx3 = hx.reshape(B, N, U)

    x = jnp.concatenate([inp3, hx3], axis=2)
    x0 = jnp.transpose(x, (1, 2, 0)).reshape(N, IN_SIZE * B)
    x0a, x0b = _split_x0(x0)
    xarr1 = _to_xarr_sc(_cheb_stack_sc(x0a, x0b, sup1, sup2))

    r, u = _ru_call(xarr1, W_ru_p, b_ru)
    r3 = r.reshape(B, N, U)

    x2nd = jnp.concatenate([inp3, r3 * hx3], axis=2)
    x0n = jnp.transpose(x2nd, (1, 2, 0)).reshape(N, IN_SIZE * B)
    x0na, x0nb = _split_x0(x0n)
    xarr2 = _to_xarr_sc(_cheb_stack_sc(x0na, x0nb, sup1, sup2))

    hx2 = hx.reshape(B * N, U)
    new_state = _gru_call(xarr2, W_c_p, b_c, u, hx2)
    return new_state.reshape(B, N * U)


# speedup vs baseline: 1.0728x; 1.0728x over previous
import functools

import jax
import jax.numpy as jnp
from jax import lax
from jax.experimental import pallas as pl
from jax.experimental.pallas import tpu as pltpu
from jax.experimental.pallas import tpu_sc as plsc

N = 10000
E = 160000
U = 64
IN_DIM = 2
K = 2
B = 8
IN_SIZE = IN_DIM + U
NUM_MAT = 2 * K + 1
FAN_IN = IN_SIZE * NUM_MAT
FAN_PAD = 384
MBLK = 800

NC = 2
NS = 16
NW = NC * NS
L = 16
NRT = 320
N_PAD = NW * NRT
FH = 272
HALF = 264
CH = 64
META = 1024
E_PAD = E + 2 * META
ROW_SENTINEL = 1 << 20


def _splits_at(splits_v, w):
    i = lax.iota(jnp.int32, L)
    total = jnp.zeros((L,), jnp.int32)
    for cidx in range(48 // L):
        ch = splits_v[pl.ds(cidx * L, L)]
        lane = w - cidx * L
        total = total + jnp.where(i == lane, ch, 0)
    return jnp.sum(total)


def _spmm_body(xa, xb, meta, splits, ya, yb,
               splits_v, mbuf, gbuf0, gbuf1, acc,
               gsem0, gsem1):
    c = lax.axis_index("c")
    s = lax.axis_index("s")
    wid = s * NC + c
    pltpu.sync_copy(splits, splits_v)
    start = _splits_at(splits_v, wid)
    end = _splits_at(splits_v, wid + 1)
    base = wid * NRT
    a_start = (start // 8) * 8
    nbig = (end - a_start + META - 1) // META
    NPAIR = META // (2 * CH)
    LASTA = (NPAIR - 1) * 2 * CH

    for tab, out in ((xa, ya), (xb, yb)):
        def zero_body(i, _):
            for j in range(FH // L):
                acc[i, pl.ds(j * L, L)] = jnp.zeros((L,), jnp.float32)
            return 0
        lax.fori_loop(0, NRT, zero_body, 0)

        def proc(gb, moff):
            def grp_body(g16, _):
                m = moff + g16 * L
                rows16 = mbuf[0, pl.ds(m, L)]
                vals16 = plsc.bitcast(mbuf[2, pl.ds(m, L)], jnp.float32)
                ok16 = (rows16 >= base) & (rows16 < base + NRT)
                v16 = jnp.where(ok16, vals16, 0.0)
                lr16 = lax.max(0, lax.min(rows16 - base, NRT - 1))

                def fast_same_row():
                    lr = lr16[0]
                    for js in (range(0, 9), range(9, FH // L)):
                        ps = {j: jnp.zeros((L,), jnp.float32) for j in js}
                        for e in range(L):
                            v = v16[e]
                            eb = g16 * L + e
                            xs = {j: gb[eb, pl.ds(j * L, L)] for j in js}
                            for j in js:
                                ps[j] = ps[j] + v * xs[j]
                        for j in js:
                            plsc.addupdate(acc.at[lr, pl.ds(j * L, L)], ps[j])

                def per_edge():
                    for e in range(L):
                        v = v16[e]
                        lr = lr16[e]
                        eb = g16 * L + e
                        xs = [gb[eb, pl.ds(j * L, L)] for j in range(FH // L)]
                        ps = [v * x for x in xs]
                        for j in range(FH // L):
                            plsc.addupdate(acc.at[lr, pl.ds(j * L, L)], ps[j])

                lax.cond(rows16[0] == rows16[L - 1], fast_same_row, per_edge)
                return 0
            lax.fori_loop(0, CH // L, grp_body, 0)

        def drain0():
            pltpu.make_async_copy(tab.at[pl.ds(0, CH)], gbuf0, gsem0).wait()

        def drain1():
            pltpu.make_async_copy(tab.at[pl.ds(0, CH)], gbuf1, gsem1).wait()

        def big_body(g, _):
            off = a_start + g * META
            pltpu.sync_copy(meta.at[:, pl.ds(off, META)], mbuf)
            pltpu.async_copy(tab.at[mbuf.at[1, pl.ds(0, CH)]], gbuf0, gsem0)
            pltpu.async_copy(tab.at[mbuf.at[1, pl.ds(CH, CH)]], gbuf1, gsem1)

            def pair_body(p, _):
                moffA = p * 2 * CH
                nextA = lax.min(moffA + 2 * CH, LASTA)
                drain0()
                proc(gbuf0, moffA)
                pltpu.async_copy(
                    tab.at[mbuf.at[1, pl.ds(nextA, CH)]], gbuf0, gsem0)
                drain1()
                proc(gbuf1, moffA + CH)
                pltpu.async_copy(
                    tab.at[mbuf.at[1, pl.ds(nextA + CH, CH)]], gbuf1, gsem1)
                return 0
            lax.fori_loop(0, NPAIR, pair_body, 0)
            drain0()
            drain1()
            return 0
        lax.fori_loop(0, nbig, big_body, 0)
        pltpu.sync_copy(acc, out.at[pl.ds(base, NRT)])


_sc_spmm = pl.kernel(
    _spmm_body,
    out_type=[jax.ShapeDtypeStruct((N_PAD, FH), jnp.float32)] * 2,
    mesh=plsc.VectorSubcoreMesh(core_axis_name="c", subcore_axis_name="s"),
    compiler_params=pltpu.CompilerParams(
        needs_layout_passes=False, use_tc_tiling_on_sc=False),
    scratch_types=[
        pltpu.VMEM((48,), jnp.int32),
        pltpu.VMEM((3, META), jnp.int32),
        pltpu.VMEM((CH, FH), jnp.float32),
        pltpu.VMEM((CH, FH), jnp.float32),
        pltpu.VMEM((NRT, FH), jnp.float32),
        pltpu.SemaphoreType.DMA,
        pltpu.SemaphoreType.DMA,
    ],
)


def _prep_support(rows, cols, vals):
    order = jnp.argsort(rows)
    rs = rows[order]
    cs = cols[order]
    vs = vals[order]
    bounds = NRT * jnp.arange(NW + 1, dtype=jnp.int32)
    splits = jnp.searchsorted(rs, bounds).astype(jnp.int32)
    splits = jnp.pad(splits, (0, 48 - (NW + 1)))
    rs = jnp.pad(rs, (0, E_PAD - E), constant_values=ROW_SENTINEL)
    cs = jnp.pad(cs, (0, E_PAD - E))
    vs = jnp.pad(vs, (0, E_PAD - E))
    meta = jnp.stack([rs, cs, vs.view(jnp.int32)], axis=0)
    return meta, splits


def _split_x0(x0):
    xa = jnp.pad(x0[:, :HALF], ((0, N_PAD - N), (0, FH - HALF)))
    xb = jnp.pad(x0[:, HALF:], ((0, N_PAD - N), (0, FH - HALF)))
    return xa, xb


def _cheb_stack_sc(x0a, x0b, sup1, sup2):
    xs = [(x0a, x0b)]
    x0 = (x0a, x0b)
    for sup in (sup1, sup2):
        x1 = _sc_spmm(x0[0], x0[1], *sup)
        xs.append(x1)
        z = _sc_spmm(x1[0], x1[1], *sup)
        x2 = (2.0 * z[0] - x0[0], 2.0 * z[1] - x0[1])
        xs.append(x2)
        x1, x0 = x2, x1
    return xs


def _to_xarr_sc(xs):
    mats = [jnp.concatenate([a[:N, :HALF], b[:N, :HALF]], axis=1)
            for (a, b) in xs]
    xarr = jnp.stack(mats, axis=0).reshape(NUM_MAT, N, IN_SIZE, B)
    return jnp.transpose(xarr, (3, 1, 2, 0)).reshape(B * N, FAN_IN)


def _ru_body(x_ref, w_ref, b_ref, r_ref, u_ref):
    acc = jnp.dot(x_ref[...], w_ref[...], preferred_element_type=jnp.float32)
    val = jax.nn.sigmoid(acc + b_ref[...])
    r_ref[...] = val[:, :U]
    u_ref[...] = val[:, U:]


def _gru_body(x_ref, w_ref, b_ref, u_ref, hx_ref, out_ref):
    acc = jnp.dot(x_ref[...], w_ref[...], preferred_element_type=jnp.float32)
    c = jnp.tanh(acc + b_ref[...])
    u = u_ref[...]
    out_ref[...] = u * hx_ref[...] + (1.0 - u) * c


def _ru_call(xarr, W, b):
    grid = (B * N) // MBLK
    return pl.pallas_call(
        _ru_body,
        grid=(grid,),
        in_specs=[
            pl.BlockSpec((MBLK, FAN_IN), lambda i: (i, 0)),
            pl.BlockSpec((FAN_IN, 2 * U), lambda i: (0, 0)),
            pl.BlockSpec((1, 2 * U), lambda i: (0, 0)),
        ],
        out_specs=[
            pl.BlockSpec((MBLK, U), lambda i: (i, 0)),
            pl.BlockSpec((MBLK, U), lambda i: (i, 0)),
        ],
        out_shape=[
            jax.ShapeDtypeStruct((B * N, U), jnp.float32),
            jax.ShapeDtypeStruct((B * N, U), jnp.float32),
        ],
    )(xarr, W, b.reshape(1, -1))


def _gru_call(xarr, W, b, u, hx):
    grid = (B * N) // MBLK
    return pl.pallas_call(
        _gru_body,
        grid=(grid,),
        in_specs=[
            pl.BlockSpec((MBLK, FAN_IN), lambda i: (i, 0)),
            pl.BlockSpec((FAN_IN, U), lambda i: (0, 0)),
            pl.BlockSpec((1, U), lambda i: (0, 0)),
            pl.BlockSpec((MBLK, U), lambda i: (i, 0)),
            pl.BlockSpec((MBLK, U), lambda i: (i, 0)),
        ],
        out_specs=pl.BlockSpec((MBLK, U), lambda i: (i, 0)),
        out_shape=jax.ShapeDtypeStruct((B * N, U), jnp.float32),
    )(xarr, W, b.reshape(1, -1), u, hx)


def kernel(inputs, hx, W_ru, b_ru, W_c, b_c,
           s1_rows, s1_cols, s1_vals, s2_rows, s2_cols, s2_vals):
    sup1 = _prep_support(s1_rows, s1_cols, s1_vals)
    sup2 = _prep_support(s2_rows, s2_cols, s2_vals)
    W_ru_p = W_ru
    W_c_p = W_c

    inp3 = inputs.reshape(B, N, IN_DIM)
    hx3 = hx.reshape(B, N, U)

    x = jnp.concatenate([inp3, hx3], axis=2)
    x0 = jnp.transpose(x, (1, 2, 0)).reshape(N, IN_SIZE * B)
    x0a, x0b = _split_x0(x0)
    xarr1 = _to_xarr_sc(_cheb_stack_sc(x0a, x0b, sup1, sup2))

    r, u = _ru_call(xarr1, W_ru_p, b_ru)
    r3 = r.reshape(B, N, U)

    x2nd = jnp.concatenate([inp3, r3 * hx3], axis=2)
    x0n = jnp.transpose(x2nd, (1, 2, 0)).reshape(N, IN_SIZE * B)
    x0na, x0nb = _split_x0(x0n)
    xarr2 = _to_xarr_sc(_cheb_stack_sc(x0na, x0nb, sup1, sup2))

    hx2 = hx.reshape(B * N, U)
    new_state = _gru_call(xarr2, W_c_p, b_c, u, hx2)
    return new_state.reshape(B, N * U)


# speedup vs baseline: 1.1353x; 1.0583x over previous
import functools

import jax
import jax.numpy as jnp
from jax import lax
from jax.experimental import pallas as pl
from jax.experimental.pallas import tpu as pltpu
from jax.experimental.pallas import tpu_sc as plsc

N = 10000
E = 160000
U = 64
IN_DIM = 2
K = 2
B = 8
IN_SIZE = IN_DIM + U
NUM_MAT = 2 * K + 1
FAN_IN = IN_SIZE * NUM_MAT
FAN_PAD = 384
MBLK = 800

NC = 2
NS = 16
NW = NC * NS
L = 16
NRT = 320
N_PAD = NW * NRT
FH = 272
HALF = 264
CH = 64
META = 2048
E_PAD = E + 2 * META
ROW_SENTINEL = 1 << 20


def _splits_at(splits_v, w):
    i = lax.iota(jnp.int32, L)
    total = jnp.zeros((L,), jnp.int32)
    for cidx in range(48 // L):
        ch = splits_v[pl.ds(cidx * L, L)]
        lane = w - cidx * L
        total = total + jnp.where(i == lane, ch, 0)
    return jnp.sum(total)


def _spmm_body(xa, xb, meta, splits, ya, yb,
               splits_v, mbuf, gbuf0, gbuf1, acc,
               gsem0, gsem1):
    c = lax.axis_index("c")
    s = lax.axis_index("s")
    wid = s * NC + c
    pltpu.sync_copy(splits, splits_v)
    start = _splits_at(splits_v, wid)
    end = _splits_at(splits_v, wid + 1)
    base = wid * NRT
    a_start = (start // 8) * 8
    nbig = (end - a_start + META - 1) // META
    NPAIR = META // (2 * CH)
    LASTA = (NPAIR - 1) * 2 * CH

    for tab, out in ((xa, ya), (xb, yb)):
        def zero_body(i, _):
            for j in range(FH // L):
                acc[i, pl.ds(j * L, L)] = jnp.zeros((L,), jnp.float32)
            return 0
        lax.fori_loop(0, NRT, zero_body, 0)

        def proc(gb, moff):
            def grp_body(g16, _):
                m = moff + g16 * L
                rows16 = mbuf[0, pl.ds(m, L)]
                vals16 = plsc.bitcast(mbuf[2, pl.ds(m, L)], jnp.float32)
                ok16 = (rows16 >= base) & (rows16 < base + NRT)
                v16 = jnp.where(ok16, vals16, 0.0)
                lr16 = lax.max(0, lax.min(rows16 - base, NRT - 1))
                for e in range(L):
                    v = v16[e]
                    lr = lr16[e]
                    eb = g16 * L + e
                    xs = [gb[eb, pl.ds(j * L, L)] for j in range(FH // L)]
                    ps = [v * x for x in xs]
                    for j in range(FH // L):
                        plsc.addupdate(acc.at[lr, pl.ds(j * L, L)], ps[j])
                return 0
            lax.fori_loop(0, CH // L, grp_body, 0)

        def drain0():
            pltpu.make_async_copy(tab.at[pl.ds(0, CH)], gbuf0, gsem0).wait()

        def drain1():
            pltpu.make_async_copy(tab.at[pl.ds(0, CH)], gbuf1, gsem1).wait()

        def big_body(g, _):
            off = a_start + g * META
            pltpu.sync_copy(meta.at[:, pl.ds(off, META)], mbuf)
            pltpu.async_copy(tab.at[mbuf.at[1, pl.ds(0, CH)]], gbuf0, gsem0)
            pltpu.async_copy(tab.at[mbuf.at[1, pl.ds(CH, CH)]], gbuf1, gsem1)

            def pair_body(p, _):
                moffA = p * 2 * CH
                nextA = lax.min(moffA + 2 * CH, LASTA)
                drain0()
                proc(gbuf0, moffA)
                pltpu.async_copy(
                    tab.at[mbuf.at[1, pl.ds(nextA, CH)]], gbuf0, gsem0)
                drain1()
                proc(gbuf1, moffA + CH)
                pltpu.async_copy(
                    tab.at[mbuf.at[1, pl.ds(nextA + CH, CH)]], gbuf1, gsem1)
                return 0
            lax.fori_loop(0, NPAIR, pair_body, 0)
            drain0()
            drain1()
            return 0
        lax.fori_loop(0, nbig, big_body, 0)
        pltpu.sync_copy(acc, out.at[pl.ds(base, NRT)])


_sc_spmm = pl.kernel(
    _spmm_body,
    out_type=[jax.ShapeDtypeStruct((N_PAD, FH), jnp.float32)] * 2,
    mesh=plsc.VectorSubcoreMesh(core_axis_name="c", subcore_axis_name="s"),
    compiler_params=pltpu.CompilerParams(
        needs_layout_passes=False, use_tc_tiling_on_sc=False),
    scratch_types=[
        pltpu.VMEM((48,), jnp.int32),
        pltpu.VMEM((3, META), jnp.int32),
        pltpu.VMEM((CH, FH), jnp.float32),
        pltpu.VMEM((CH, FH), jnp.float32),
        pltpu.VMEM((NRT, FH), jnp.float32),
        pltpu.SemaphoreType.DMA,
        pltpu.SemaphoreType.DMA,
    ],
)


def _prep_support(rows, cols, vals):
    rs, cs, vs = lax.sort((rows, cols, vals), num_keys=1)
    bounds = NRT * jnp.arange(NW + 1, dtype=jnp.int32)
    splits = jnp.searchsorted(rs, bounds).astype(jnp.int32)
    splits = jnp.pad(splits, (0, 48 - (NW + 1)))
    rs = jnp.pad(rs, (0, E_PAD - E), constant_values=ROW_SENTINEL)
    cs = jnp.pad(cs, (0, E_PAD - E))
    vs = jnp.pad(vs, (0, E_PAD - E))
    meta = jnp.stack([rs, cs, vs.view(jnp.int32)], axis=0)
    return meta, splits


def _split_x0(x0):
    xa = jnp.pad(x0[:, :HALF], ((0, N_PAD - N), (0, FH - HALF)))
    xb = jnp.pad(x0[:, HALF:], ((0, N_PAD - N), (0, FH - HALF)))
    return xa, xb


def _cheb_stack_sc(x0a, x0b, sup1, sup2):
    xs = [(x0a, x0b)]
    x0 = (x0a, x0b)
    for sup in (sup1, sup2):
        x1 = _sc_spmm(x0[0], x0[1], *sup)
        xs.append(x1)
        z = _sc_spmm(x1[0], x1[1], *sup)
        x2 = (2.0 * z[0] - x0[0], 2.0 * z[1] - x0[1])
        xs.append(x2)
        x1, x0 = x2, x1
    return xs


def _to_xarr_sc(xs):
    mats = [jnp.concatenate([a[:N, :HALF], b[:N, :HALF]], axis=1)
            for (a, b) in xs]
    xarr = jnp.stack(mats, axis=0).reshape(NUM_MAT, N, IN_SIZE, B)
    return jnp.transpose(xarr, (3, 1, 2, 0)).reshape(B * N, FAN_IN)


def _ru_body(x_ref, w_ref, b_ref, r_ref, u_ref):
    acc = jnp.dot(x_ref[...], w_ref[...], preferred_element_type=jnp.float32)
    val = jax.nn.sigmoid(acc + b_ref[...])
    r_ref[...] = val[:, :U]
    u_ref[...] = val[:, U:]


def _gru_body(x_ref, w_ref, b_ref, u_ref, hx_ref, out_ref):
    acc = jnp.dot(x_ref[...], w_ref[...], preferred_element_type=jnp.float32)
    c = jnp.tanh(acc + b_ref[...])
    u = u_ref[...]
    out_ref[...] = u * hx_ref[...] + (1.0 - u) * c


def _ru_call(xarr, W, b):
    grid = (B * N) // MBLK
    return pl.pallas_call(
        _ru_body,
        grid=(grid,),
        in_specs=[
            pl.BlockSpec((MBLK, FAN_IN), lambda i: (i, 0)),
            pl.BlockSpec((FAN_IN, 2 * U), lambda i: (0, 0)),
            pl.BlockSpec((1, 2 * U), lambda i: (0, 0)),
        ],
        out_specs=[
            pl.BlockSpec((MBLK, U), lambda i: (i, 0)),
            pl.BlockSpec((MBLK, U), lambda i: (i, 0)),
        ],
        out_shape=[
            jax.ShapeDtypeStruct((B * N, U), jnp.float32),
            jax.ShapeDtypeStruct((B * N, U), jnp.float32),
        ],
    )(xarr, W, b.reshape(1, -1))


def _gru_call(xarr, W, b, u, hx):
    grid = (B * N) // MBLK
    return pl.pallas_call(
        _gru_body,
        grid=(grid,),
        in_specs=[
            pl.BlockSpec((MBLK, FAN_IN), lambda i: (i, 0)),
            pl.BlockSpec((FAN_IN, U), lambda i: (0, 0)),
            pl.BlockSpec((1, U), lambda i: (0, 0)),
            pl.BlockSpec((MBLK, U), lambda i: (i, 0)),
            pl.BlockSpec((MBLK, U), lambda i: (i, 0)),
        ],
        out_specs=pl.BlockSpec((MBLK, U), lambda i: (i, 0)),
        out_shape=jax.ShapeDtypeStruct((B * N, U), jnp.float32),
    )(xarr, W, b.reshape(1, -1), u, hx)


def kernel(inputs, hx, W_ru, b_ru, W_c, b_c,
           s1_rows, s1_cols, s1_vals, s2_rows, s2_cols, s2_vals):
    sup1 = _prep_support(s1_rows, s1_cols, s1_vals)
    sup2 = _prep_support(s2_rows, s2_cols, s2_vals)
    W_ru_p = W_ru
    W_c_p = W_c

    inp3 = inputs.reshape(B, N, IN_DIM)
    hx3 = hx.reshape(B, N, U)

    x = jnp.concatenate([inp3, hx3], axis=2)
    x0 = jnp.transpose(x, (1, 2, 0)).reshape(N, IN_SIZE * B)
    x0a, x0b = _split_x0(x0)
    xarr1 = _to_xarr_sc(_cheb_stack_sc(x0a, x0b, sup1, sup2))

    r, u = _ru_call(xarr1, W_ru_p, b_ru)
    r3 = r.reshape(B, N, U)

    x2nd = jnp.concatenate([inp3, r3 * hx3], axis=2)
    x0n = jnp.transpose(x2nd, (1, 2, 0)).reshape(N, IN_SIZE * B)
    x0na, x0nb = _split_x0(x0n)
    xarr2 = _to_xarr_sc(_cheb_stack_sc(x0na, x0nb, sup1, sup2))

    hx2 = hx.reshape(B * N, U)
    new_state = _gru_call(xarr2, W_c_p, b_c, u, hx2)
    return new_state.reshape(B, N * U)
